# baseline (speedup 1.0000x reference)
import functools

import jax
import jax.numpy as jnp
from jax.experimental import pallas as pl
from jax.experimental.pallas import tpu as pltpu


def _se_kernel(x_ref, pool_ref, w1_ref, w2_ref, o_ref, sc_ref, *, c):
    bc, hw = x_ref.shape
    b = bc // c
    reps = hw // 128
    sc_ref[...] = jnp.dot(x_ref[...], pool_ref[...],
                          preferred_element_type=jnp.float32)
    for i in range(b):
        s = sc_ref[i * c:(i + 1) * c, :]
        h = jnp.dot(w1_ref[...], s, preferred_element_type=jnp.float32)
        h = h * jax.nn.sigmoid(h)
        g = jax.nn.sigmoid(
            jnp.dot(w2_ref[...], h, preferred_element_type=jnp.float32))
        gg = jnp.concatenate([g] * reps, axis=1) if reps > 1 else g
        o_ref[i * c:(i + 1) * c, :] = x_ref[i * c:(i + 1) * c, :] * gg


def kernel(x_nchw, w1, w2):
    n, c, h, w = x_nchw.shape
    hw = h * w
    cr = w1.shape[0]
    dtype = x_nchw.dtype
    itemsize = dtype.itemsize

    x2 = x_nchw.reshape(n * c, hw)
    pool = jnp.full((hw, 128), 1.0 / hw, dtype=jnp.float32)
    w1f = w1.astype(jnp.float32)
    w2f = w2.astype(jnp.float32)

    b_tile = min(n, 16)
    while n % b_tile:
        b_tile -= 1
    num_blocks = n // b_tile
    rows = b_tile * c

    block_bytes = rows * hw * itemsize
    vmem_limit = int(min(48 * 1024 * 1024,
                         4 * block_bytes + rows * 128 * 4 + 4 * 1024 * 1024))
    cost = pl.CostEstimate(
        flops=2 * n * c * hw * 128 + 4 * n * c * cr * 128 + n * c * hw,
        transcendentals=3 * n * (c + cr) * 128,
        bytes_accessed=2 * n * c * hw * itemsize + (2 * c * cr + hw * 128) * 4,
    )
    out = pl.pallas_call(
        functools.partial(_se_kernel, c=c),
        out_shape=jax.ShapeDtypeStruct((n * c, hw), dtype),
        grid=(num_blocks,),
        in_specs=[
            pl.BlockSpec((rows, hw), lambda i: (i, 0)),
            pl.BlockSpec((hw, 128), lambda i: (0, 0)),
            pl.BlockSpec((cr, c), lambda i: (0, 0)),
            pl.BlockSpec((c, cr), lambda i: (0, 0)),
        ],
        out_specs=pl.BlockSpec((rows, hw), lambda i: (i, 0)),
        scratch_shapes=[pltpu.VMEM((rows, 128), jnp.float32)],
        compiler_params=pltpu.CompilerParams(
            dimension_semantics=("parallel",),
            vmem_limit_bytes=vmem_limit),
        cost_estimate=cost,
    )(x2, pool, w1f, w2f)
    return out.reshape(n, c, h, w)


# speedup vs baseline: 8.2031x; 8.2031x over previous
import functools

import jax
import jax.numpy as jnp
from jax.experimental import pallas as pl
from jax.experimental.pallas import tpu as pltpu


def _se_kernel(x_ref, w1_ref, w2_ref, o_ref, *, inv_hw):
    b = x_ref.shape[0]
    dims = (((1,), (1,)), ((), ()))
    for i in range(b):
        s = jnp.sum(x_ref[i], axis=0, keepdims=True) * inv_hw
        h = jax.lax.dot_general(s, w1_ref[...], dims,
                                preferred_element_type=jnp.float32)
        h = h * jax.nn.sigmoid(h)
        g = jax.nn.sigmoid(
            jax.lax.dot_general(h, w2_ref[...], dims,
                                preferred_element_type=jnp.float32))
        o_ref[i] = x_ref[i] * g


def kernel(x_nchw, w1, w2):
    n, c, h, w = x_nchw.shape
    hw = h * w
    cr = w1.shape[0]
    dtype = x_nchw.dtype
    itemsize = dtype.itemsize

    x_nhwc = jnp.transpose(x_nchw, (0, 2, 3, 1)).reshape(n, hw, c)
    w1f = w1.astype(jnp.float32)
    w2f = w2.astype(jnp.float32)

    b_tile = min(n, 16)
    while n % b_tile:
        b_tile -= 1
    num_blocks = n // b_tile

    block_bytes = b_tile * hw * c * itemsize
    vmem_limit = int(min(48 * 1024 * 1024,
                         4 * block_bytes + 4 * 1024 * 1024))
    cost = pl.CostEstimate(
        flops=3 * n * c * hw + 4 * n * c * cr,
        transcendentals=3 * n * (c + cr),
        bytes_accessed=2 * n * c * hw * itemsize + 2 * c * cr * 4,
    )
    out = pl.pallas_call(
        functools.partial(_se_kernel, inv_hw=1.0 / hw),
        out_shape=jax.ShapeDtypeStruct((n, hw, c), dtype),
        grid=(num_blocks,),
        in_specs=[
            pl.BlockSpec((b_tile, hw, c), lambda i: (i, 0, 0)),
            pl.BlockSpec((cr, c), lambda i: (0, 0)),
            pl.BlockSpec((c, cr), lambda i: (0, 0)),
        ],
        out_specs=pl.BlockSpec((b_tile, hw, c), lambda i: (i, 0, 0)),
        compiler_params=pltpu.CompilerParams(
            dimension_semantics=("parallel",),
            vmem_limit_bytes=vmem_limit),
        cost_estimate=cost,
    )(x_nhwc, w1f, w2f)
    return jnp.transpose(out.reshape(n, h, w, c), (0, 3, 1, 2))
